# baseline (speedup 1.0000x reference)
import functools

import jax
import jax.numpy as jnp
import numpy as np
from jax import lax
from jax.experimental import pallas as pl
from jax.experimental.pallas import tpu as pltpu
from jax.experimental.pallas import tpu_sc as plsc

NUM_THETAS = 16
NUM_PHIS = 16
BUMP_STEPS = 16
RADIUS = 1.1
N_GRAPHS = 8
D = NUM_THETAS * NUM_PHIS
SD = BUMP_STEPS * D
ACC = N_GRAPHS * SD

BLK_N = 4096

N_SC = 4096
NW = 32
CHUNK = N_SC // NW

_LIN = np.linspace(-RADIUS, RADIUS, BUMP_STEPS).astype(np.float32)
_C200 = [float(200.0 * _LIN[s]) for s in range(BUMP_STEPS)]


def _directions():
    theta = jnp.linspace(0.0, jnp.pi, NUM_THETAS)
    phi = jnp.linspace(0.0, 2.0 * jnp.pi, NUM_PHIS)
    mt, mp = jnp.meshgrid(theta, phi, indexing="ij")
    v = jnp.stack(
        [
            (jnp.sin(mt) * jnp.cos(mp)).reshape(-1),
            (jnp.sin(mt) * jnp.sin(mp)).reshape(-1),
            jnp.cos(mt).reshape(-1),
        ],
        axis=0,
    )
    return v.astype(jnp.float32)


def _tc_kernel(xt_ref, batch_ref, v_ref, out_ref):
    @pl.when(pl.program_id(0) == 0)
    def _init():
        out_ref[...] = jnp.zeros_like(out_ref)

    nh100 = jax.lax.dot_general(
        xt_ref[...], v_ref[...], (((0,), (0,)), ((), ())),
        preferred_element_type=jnp.float32,
    )

    parts = []
    for s in range(BUMP_STEPS):
        arg = float(100.0 * _LIN[s]) - nh100
        parts.append(jnp.tanh(arg.astype(jnp.bfloat16)))
    tanh_all = jnp.concatenate(parts, axis=1)

    b_ids = lax.broadcasted_iota(jnp.int32, (N_GRAPHS, BLK_N), 0)
    onehot = (b_ids == batch_ref[0]).astype(jnp.bfloat16)
    seg = jax.lax.dot_general(
        onehot, tanh_all, (((1,), (0,)), ((), ())),
        preferred_element_type=jnp.float32,
    )
    count = jnp.sum(onehot.astype(jnp.float32), axis=1, keepdims=True)
    out_ref[...] += 0.5 * seg + 0.5 * count


def _tc_part(x, batch):
    n = x.shape[0]
    xt = jnp.zeros((8, n), dtype=jnp.float32).at[:3, :].set(x.T)
    v = jnp.zeros((8, D), dtype=jnp.float32).at[:3, :].set(100.0 * _directions())
    nblk = n // BLK_N
    batch3 = batch.reshape(nblk, 1, BLK_N)
    return pl.pallas_call(
        _tc_kernel,
        grid=(nblk,),
        in_specs=[
            pl.BlockSpec((8, BLK_N), lambda g: (0, g)),
            pl.BlockSpec((1, 1, BLK_N), lambda g: (g, 0, 0)),
            pl.BlockSpec((8, D), lambda g: (0, 0)),
        ],
        out_specs=pl.BlockSpec((N_GRAPHS, SD), lambda g: (0, 0)),
        out_shape=jax.ShapeDtypeStruct((N_GRAPHS, SD), jnp.float32),
    )(xt, batch3, v)


def _sc_body(xf_hbm, b_hbm, w_hbm, out_hbm, x_v, b_v, w_v, acc_v, slab,
             tmp_v, red_v):
    cid = lax.axis_index("c")
    sid = lax.axis_index("s")
    wid = sid * 2 + cid
    base = wid * CHUNK

    pltpu.sync_copy(xf_hbm.at[pl.ds(base, CHUNK)], x_v.at[pl.ds(0, CHUNK)])
    pltpu.sync_copy(
        xf_hbm.at[pl.ds(N_SC + base, CHUNK)], x_v.at[pl.ds(CHUNK, CHUNK)])
    pltpu.sync_copy(
        xf_hbm.at[pl.ds(2 * N_SC + base, CHUNK)], x_v.at[pl.ds(2 * CHUNK, CHUNK)])
    pltpu.sync_copy(b_hbm.at[pl.ds(base, CHUNK)], b_v)
    pltpu.sync_copy(w_hbm, w_v)

    for r in range(16):
        def zero_body(j, _, r=r):
            acc_v[r, pl.ds(pl.multiple_of(j * 16, 16), 16)] = jnp.zeros(
                (16,), jnp.float32)
            return 0

        lax.fori_loop(0, ACC // 256, zero_body, 0)

    lanes = lax.iota(jnp.int32, 16)

    def point_body(i, _):
        iv = jnp.full((16,), i, jnp.int32)
        x0 = plsc.load_gather(x_v, [iv])
        x1 = plsc.load_gather(x_v, [iv + CHUNK])
        x2 = plsc.load_gather(x_v, [iv + 2 * CHUNK])
        bb = plsc.load_gather(b_v, [iv])
        row0 = bb * 2
        for c in range(D // 16):
            w0 = w_v[pl.ds(c * 16, 16)]
            w1 = w_v[pl.ds(D + c * 16, 16)]
            w2 = w_v[pl.ds(2 * D + c * 16, 16)]
            t = x0 * w0 + x1 * w1 + x2 * w2
            col0 = lanes + (c * 16)
            for s in range(BUMP_STEPS):
                sig = 1.0 / (1.0 + jnp.exp(t - _C200[s]))
                plsc.addupdate_scatter(
                    acc_v, [row0 + (s // 8), col0 + (s % 8) * D], sig)
        return 0

    lax.fori_loop(0, CHUNK, point_body, 0)

    pltpu.sync_copy(acc_v, slab.at[sid])
    plsc.subcore_barrier()
    for t in range(16):
        pltpu.sync_copy(slab.at[t, sid], tmp_v.at[t])
    for j in range(ACC // 256):
        sl = pl.ds(j * 16, 16)
        s = tmp_v[0, sl]
        for t in range(1, 16):
            s = s + tmp_v[t, sl]
        red_v[sl] = s
    pltpu.sync_copy(red_v, out_hbm.at[cid, sid])


def _sc_part(x_sc, b_sc):
    xf = (200.0 * x_sc.T).reshape(-1)
    w = _directions().reshape(-1)
    run = functools.partial(
        pl.kernel,
        out_type=jax.ShapeDtypeStruct((2, 16, ACC // 16), jnp.float32),
        mesh=plsc.VectorSubcoreMesh(core_axis_name="c", subcore_axis_name="s"),
        scratch_types=[
            pltpu.VMEM((3 * CHUNK,), jnp.float32),
            pltpu.VMEM((CHUNK,), jnp.int32),
            pltpu.VMEM((3 * D,), jnp.float32),
            pltpu.VMEM((16, ACC // 16), jnp.float32),
            pltpu.VMEM_SHARED((16, 16, ACC // 16), jnp.float32),
            pltpu.VMEM((16, ACC // 16), jnp.float32),
            pltpu.VMEM((ACC // 16,), jnp.float32),
        ],
        compiler_params=pltpu.CompilerParams(needs_layout_passes=False),
    )(_sc_body)
    return run(xf, b_sc, w)


def kernel(x, batch):
    n = x.shape[0]
    n_tc = n - N_SC
    tc_out = _tc_part(x[:n_tc], batch[:n_tc])
    sc_out = _sc_part(x[n_tc:], batch[n_tc:])
    sc_sum = (sc_out[0] + sc_out[1]).reshape(N_GRAPHS, SD)
    total = tc_out + sc_sum
    return total.reshape(N_GRAPHS, BUMP_STEPS, NUM_THETAS, NUM_PHIS)


# speedup vs baseline: 8.5460x; 8.5460x over previous
import functools

import jax
import jax.numpy as jnp
import numpy as np
from jax import lax
from jax.experimental import pallas as pl
from jax.experimental.pallas import tpu as pltpu
from jax.experimental.pallas import tpu_sc as plsc

NUM_THETAS = 16
NUM_PHIS = 16
BUMP_STEPS = 16
RADIUS = 1.1
N_GRAPHS = 8
D = NUM_THETAS * NUM_PHIS
SD = BUMP_STEPS * D
ACC = N_GRAPHS * SD

BLK_N = 4096

N_SC = 4096
NW = 32
CHUNK = N_SC // NW

_LIN = np.linspace(-RADIUS, RADIUS, BUMP_STEPS).astype(np.float32)
_C200 = [float(200.0 * _LIN[s]) for s in range(BUMP_STEPS)]


def _directions():
    theta = jnp.linspace(0.0, jnp.pi, NUM_THETAS)
    phi = jnp.linspace(0.0, 2.0 * jnp.pi, NUM_PHIS)
    mt, mp = jnp.meshgrid(theta, phi, indexing="ij")
    v = jnp.stack(
        [
            (jnp.sin(mt) * jnp.cos(mp)).reshape(-1),
            (jnp.sin(mt) * jnp.sin(mp)).reshape(-1),
            jnp.cos(mt).reshape(-1),
        ],
        axis=0,
    )
    return v.astype(jnp.float32)


def _tc_kernel(xt_ref, batch_ref, v_ref, out_ref):
    @pl.when(pl.program_id(0) == 0)
    def _init():
        out_ref[...] = jnp.zeros_like(out_ref)

    nh100 = jax.lax.dot_general(
        xt_ref[...], v_ref[...], (((0,), (0,)), ((), ())),
        preferred_element_type=jnp.float32,
    )

    parts = []
    for s in range(BUMP_STEPS):
        arg = float(100.0 * _LIN[s]) - nh100
        parts.append(jnp.tanh(arg.astype(jnp.bfloat16)))
    tanh_all = jnp.concatenate(parts, axis=1)

    b_ids = lax.broadcasted_iota(jnp.int32, (N_GRAPHS, BLK_N), 0)
    onehot = (b_ids == batch_ref[0]).astype(jnp.bfloat16)
    seg = jax.lax.dot_general(
        onehot, tanh_all, (((1,), (0,)), ((), ())),
        preferred_element_type=jnp.float32,
    )
    count = jnp.sum(onehot.astype(jnp.float32), axis=1, keepdims=True)
    out_ref[...] += 0.5 * seg + 0.5 * count


def _tc_part(x, batch):
    n = x.shape[0]
    xt = jnp.zeros((8, n), dtype=jnp.float32).at[:3, :].set(x.T)
    v = jnp.zeros((8, D), dtype=jnp.float32).at[:3, :].set(100.0 * _directions())
    nblk = n // BLK_N
    batch3 = batch.reshape(nblk, 1, BLK_N)
    return pl.pallas_call(
        _tc_kernel,
        grid=(nblk,),
        in_specs=[
            pl.BlockSpec((8, BLK_N), lambda g: (0, g)),
            pl.BlockSpec((1, 1, BLK_N), lambda g: (g, 0, 0)),
            pl.BlockSpec((8, D), lambda g: (0, 0)),
        ],
        out_specs=pl.BlockSpec((N_GRAPHS, SD), lambda g: (0, 0)),
        out_shape=jax.ShapeDtypeStruct((N_GRAPHS, SD), jnp.float32),
    )(xt, batch3, v)


def _sc_body(xf_hbm, b_hbm, w_hbm, out_hbm, x_v, b_v, w_v, acc_v, slab,
             tmp_v, red_v):
    cid = lax.axis_index("c")
    sid = lax.axis_index("s")
    wid = sid * 2 + cid
    base = wid * CHUNK

    pltpu.sync_copy(xf_hbm.at[pl.ds(base, CHUNK)], x_v.at[pl.ds(0, CHUNK)])
    pltpu.sync_copy(
        xf_hbm.at[pl.ds(N_SC + base, CHUNK)], x_v.at[pl.ds(CHUNK, CHUNK)])
    pltpu.sync_copy(
        xf_hbm.at[pl.ds(2 * N_SC + base, CHUNK)], x_v.at[pl.ds(2 * CHUNK, CHUNK)])
    pltpu.sync_copy(b_hbm.at[pl.ds(base, CHUNK)], b_v)
    pltpu.sync_copy(w_hbm, w_v)

    for r in range(16):
        def zero_body(j, _, r=r):
            acc_v[r, pl.ds(pl.multiple_of(j * 16, 16), 16)] = jnp.zeros(
                (16,), jnp.float32)
            return 0

        lax.fori_loop(0, ACC // 256, zero_body, 0)

    lanes = lax.iota(jnp.int32, 16)

    @functools.partial(plsc.parallel_loop, 0, CHUNK, unroll=2)
    def point_body(i):
        iv = jnp.full((16,), i, jnp.int32)
        x0 = plsc.load_gather(x_v, [iv])
        x1 = plsc.load_gather(x_v, [iv + CHUNK])
        x2 = plsc.load_gather(x_v, [iv + 2 * CHUNK])
        bb = plsc.load_gather(b_v, [iv])
        row0 = bb * 2
        for c in range(D // 16):
            w0 = w_v[pl.ds(c * 16, 16)]
            w1 = w_v[pl.ds(D + c * 16, 16)]
            w2 = w_v[pl.ds(2 * D + c * 16, 16)]
            t = x0 * w0 + x1 * w1 + x2 * w2
            col0 = lanes + (c * 16)
            for s in range(BUMP_STEPS):
                sig = 1.0 / (1.0 + jnp.exp(t - _C200[s]))
                plsc.addupdate_scatter(
                    acc_v, [row0 + (s // 8), col0 + (s % 8) * D], sig)

    pltpu.sync_copy(acc_v, slab.at[sid])
    plsc.subcore_barrier()
    for t in range(16):
        pltpu.sync_copy(slab.at[t, sid], tmp_v.at[t])
    for j in range(ACC // 256):
        sl = pl.ds(j * 16, 16)
        s = tmp_v[0, sl]
        for t in range(1, 16):
            s = s + tmp_v[t, sl]
        red_v[sl] = s
    pltpu.sync_copy(red_v, out_hbm.at[cid, sid])


def _sc_part(x_sc, b_sc):
    xf = (200.0 * x_sc.T).reshape(-1)
    w = _directions().reshape(-1)
    run = functools.partial(
        pl.kernel,
        out_type=jax.ShapeDtypeStruct((2, 16, ACC // 16), jnp.float32),
        mesh=plsc.VectorSubcoreMesh(core_axis_name="c", subcore_axis_name="s"),
        scratch_types=[
            pltpu.VMEM((3 * CHUNK,), jnp.float32),
            pltpu.VMEM((CHUNK,), jnp.int32),
            pltpu.VMEM((3 * D,), jnp.float32),
            pltpu.VMEM((16, ACC // 16), jnp.float32),
            pltpu.VMEM_SHARED((16, 16, ACC // 16), jnp.float32),
            pltpu.VMEM((16, ACC // 16), jnp.float32),
            pltpu.VMEM((ACC // 16,), jnp.float32),
        ],
        compiler_params=pltpu.CompilerParams(needs_layout_passes=False),
    )(_sc_body)
    return run(xf, b_sc, w)


def kernel(x, batch):
    n = x.shape[0]
    n_tc = n - N_SC
    tc_out = _tc_part(x[:n_tc], batch[:n_tc])
    sc_out = _sc_part(x[n_tc:], batch[n_tc:])
    sc_sum = (sc_out[0] + sc_out[1]).reshape(N_GRAPHS, SD)
    total = tc_out + sc_sum
    return total.reshape(N_GRAPHS, BUMP_STEPS, NUM_THETAS, NUM_PHIS)


# speedup vs baseline: 8.5599x; 1.0016x over previous
import functools

import jax
import jax.numpy as jnp
import numpy as np
from jax import lax
from jax.experimental import pallas as pl
from jax.experimental.pallas import tpu as pltpu
from jax.experimental.pallas import tpu_sc as plsc

NUM_THETAS = 16
NUM_PHIS = 16
BUMP_STEPS = 16
RADIUS = 1.1
N_GRAPHS = 8
D = NUM_THETAS * NUM_PHIS
SD = BUMP_STEPS * D
ACC = N_GRAPHS * SD

BLK_N = 4096

N_SC = 4096
NW = 32
CHUNK = N_SC // NW

_LIN = np.linspace(-RADIUS, RADIUS, BUMP_STEPS).astype(np.float32)
_C200 = [float(200.0 * _LIN[s]) for s in range(BUMP_STEPS)]


def _directions():
    theta = jnp.linspace(0.0, jnp.pi, NUM_THETAS)
    phi = jnp.linspace(0.0, 2.0 * jnp.pi, NUM_PHIS)
    mt, mp = jnp.meshgrid(theta, phi, indexing="ij")
    v = jnp.stack(
        [
            (jnp.sin(mt) * jnp.cos(mp)).reshape(-1),
            (jnp.sin(mt) * jnp.sin(mp)).reshape(-1),
            jnp.cos(mt).reshape(-1),
        ],
        axis=0,
    )
    return v.astype(jnp.float32)


def _tc_kernel(xt_ref, batch_ref, v_ref, out_ref):
    @pl.when(pl.program_id(0) == 0)
    def _init():
        out_ref[...] = jnp.zeros_like(out_ref)

    nh100 = jax.lax.dot_general(
        xt_ref[...], v_ref[...], (((0,), (0,)), ((), ())),
        preferred_element_type=jnp.float32,
    )

    parts = []
    for s in range(BUMP_STEPS):
        arg = float(100.0 * _LIN[s]) - nh100
        parts.append(jnp.tanh(arg.astype(jnp.bfloat16)))
    tanh_all = jnp.concatenate(parts, axis=1)

    b_ids = lax.broadcasted_iota(jnp.int32, (N_GRAPHS, BLK_N), 0)
    onehot = (b_ids == batch_ref[0]).astype(jnp.bfloat16)
    seg = jax.lax.dot_general(
        onehot, tanh_all, (((1,), (0,)), ((), ())),
        preferred_element_type=jnp.float32,
    )
    count = jnp.sum(onehot.astype(jnp.float32), axis=1, keepdims=True)
    out_ref[...] += 0.5 * seg + 0.5 * count


def _tc_part(x, batch):
    n = x.shape[0]
    xt = jnp.zeros((8, n), dtype=jnp.float32).at[:3, :].set(x.T)
    v = jnp.zeros((8, D), dtype=jnp.float32).at[:3, :].set(100.0 * _directions())
    nblk = n // BLK_N
    batch3 = batch.reshape(nblk, 1, BLK_N)
    return pl.pallas_call(
        _tc_kernel,
        grid=(nblk,),
        in_specs=[
            pl.BlockSpec((8, BLK_N), lambda g: (0, g)),
            pl.BlockSpec((1, 1, BLK_N), lambda g: (g, 0, 0)),
            pl.BlockSpec((8, D), lambda g: (0, 0)),
        ],
        out_specs=pl.BlockSpec((N_GRAPHS, SD), lambda g: (0, 0)),
        out_shape=jax.ShapeDtypeStruct((N_GRAPHS, SD), jnp.float32),
    )(xt, batch3, v)


def _sc_body(xf_hbm, b_hbm, w_hbm, out_hbm, x_v, b_v, w_v, acc_v, slab,
             tmp_v, red_v):
    cid = lax.axis_index("c")
    sid = lax.axis_index("s")
    wid = sid * 2 + cid
    base = wid * CHUNK

    pltpu.sync_copy(xf_hbm.at[pl.ds(base, CHUNK)], x_v.at[pl.ds(0, CHUNK)])
    pltpu.sync_copy(
        xf_hbm.at[pl.ds(N_SC + base, CHUNK)], x_v.at[pl.ds(CHUNK, CHUNK)])
    pltpu.sync_copy(
        xf_hbm.at[pl.ds(2 * N_SC + base, CHUNK)], x_v.at[pl.ds(2 * CHUNK, CHUNK)])
    pltpu.sync_copy(b_hbm.at[pl.ds(base, CHUNK)], b_v)
    pltpu.sync_copy(w_hbm, w_v)

    for r in range(16):
        def zero_body(j, _, r=r):
            acc_v[r, pl.ds(pl.multiple_of(j * 16, 16), 16)] = jnp.zeros(
                (16,), jnp.float32)
            return 0

        lax.fori_loop(0, ACC // 256, zero_body, 0)

    lanes = lax.iota(jnp.int32, 16)

    def point_body(i, _):
        iv = jnp.full((16,), i, jnp.int32)
        x0 = plsc.load_gather(x_v, [iv])
        x1 = plsc.load_gather(x_v, [iv + CHUNK])
        x2 = plsc.load_gather(x_v, [iv + 2 * CHUNK])
        bb = plsc.load_gather(b_v, [iv])
        row0 = bb * 2

        @functools.partial(plsc.parallel_loop, 0, D // 16, unroll=4)
        def cbody(c):
            c16 = c * 16
            w0 = w_v[pl.ds(pl.multiple_of(c16, 16), 16)]
            w1 = w_v[pl.ds(pl.multiple_of(D + c16, 16), 16)]
            w2 = w_v[pl.ds(pl.multiple_of(2 * D + c16, 16), 16)]
            t = x0 * w0 + x1 * w1 + x2 * w2
            col0 = lanes + c16
            for s in range(BUMP_STEPS):
                sig = 1.0 / (1.0 + jnp.exp(t - _C200[s]))
                plsc.addupdate_scatter(
                    acc_v, [row0 + (s // 8), col0 + (s % 8) * D], sig)
        return 0

    lax.fori_loop(0, CHUNK, point_body, 0)

    pltpu.sync_copy(acc_v, slab.at[sid])
    plsc.subcore_barrier()
    for t in range(16):
        pltpu.sync_copy(slab.at[t, sid], tmp_v.at[t])
    for j in range(ACC // 256):
        sl = pl.ds(j * 16, 16)
        s = tmp_v[0, sl]
        for t in range(1, 16):
            s = s + tmp_v[t, sl]
        red_v[sl] = s
    pltpu.sync_copy(red_v, out_hbm.at[cid, sid])


def _sc_part(x_sc, b_sc):
    xf = (200.0 * x_sc.T).reshape(-1)
    w = _directions().reshape(-1)
    run = functools.partial(
        pl.kernel,
        out_type=jax.ShapeDtypeStruct((2, 16, ACC // 16), jnp.float32),
        mesh=plsc.VectorSubcoreMesh(core_axis_name="c", subcore_axis_name="s"),
        scratch_types=[
            pltpu.VMEM((3 * CHUNK,), jnp.float32),
            pltpu.VMEM((CHUNK,), jnp.int32),
            pltpu.VMEM((3 * D,), jnp.float32),
            pltpu.VMEM((16, ACC // 16), jnp.float32),
            pltpu.VMEM_SHARED((16, 16, ACC // 16), jnp.float32),
            pltpu.VMEM((16, ACC // 16), jnp.float32),
            pltpu.VMEM((ACC // 16,), jnp.float32),
        ],
        compiler_params=pltpu.CompilerParams(needs_layout_passes=False),
    )(_sc_body)
    return run(xf, b_sc, w)


def kernel(x, batch):
    n = x.shape[0]
    n_tc = n - N_SC
    tc_out = _tc_part(x[:n_tc], batch[:n_tc])
    sc_out = _sc_part(x[n_tc:], batch[n_tc:])
    sc_sum = (sc_out[0] + sc_out[1]).reshape(N_GRAPHS, SD)
    total = tc_out + sc_sum
    return total.reshape(N_GRAPHS, BUMP_STEPS, NUM_THETAS, NUM_PHIS)


# speedup vs baseline: 10.4285x; 1.2183x over previous
import jax
import jax.numpy as jnp
import numpy as np
from jax.experimental import pallas as pl
from jax.experimental.pallas import tpu as pltpu

NUM_THETAS = 16
NUM_PHIS = 16
BUMP_STEPS = 16
RADIUS = 1.1
N_GRAPHS = 8
D = NUM_THETAS * NUM_PHIS
SD = BUMP_STEPS * D

BLK_N = 8192

_LIN = np.linspace(-RADIUS, RADIUS, BUMP_STEPS).astype(np.float32)


def _directions():
    theta = jnp.linspace(0.0, jnp.pi, NUM_THETAS)
    phi = jnp.linspace(0.0, 2.0 * jnp.pi, NUM_PHIS)
    mt, mp = jnp.meshgrid(theta, phi, indexing="ij")
    v = jnp.stack(
        [
            (jnp.sin(mt) * jnp.cos(mp)).reshape(-1),
            (jnp.sin(mt) * jnp.sin(mp)).reshape(-1),
            jnp.cos(mt).reshape(-1),
        ],
        axis=0,
    )
    return v.astype(jnp.float32)


def _fused_kernel(xt_ref, batch_ref, v_ref, out_ref):
    @pl.when(pl.program_id(0) == 0)
    def _init():
        out_ref[...] = jnp.zeros_like(out_ref)

    nh100 = jax.lax.dot_general(
        xt_ref[...], v_ref[...], (((0,), (0,)), ((), ())),
        preferred_element_type=jnp.float32,
    )

    parts = []
    for s in range(BUMP_STEPS):
        arg = float(100.0 * _LIN[s]) - nh100
        parts.append(jnp.tanh(arg.astype(jnp.bfloat16)))
    tanh_all = jnp.concatenate(parts, axis=1)

    b_ids = jax.lax.broadcasted_iota(jnp.int32, (N_GRAPHS, BLK_N), 0)
    onehot = (b_ids == batch_ref[0]).astype(jnp.bfloat16)
    seg = jax.lax.dot_general(
        onehot, tanh_all, (((1,), (0,)), ((), ())),
        preferred_element_type=jnp.float32,
    )
    count = jnp.sum(onehot.astype(jnp.float32), axis=1, keepdims=True)
    out_ref[...] += 0.5 * seg + 0.5 * count


def kernel(x, batch):
    n = x.shape[0]
    xt = jnp.zeros((8, n), dtype=jnp.float32).at[:3, :].set(x.T)
    v = jnp.zeros((8, D), dtype=jnp.float32).at[:3, :].set(100.0 * _directions())
    nblk = n // BLK_N
    batch3 = batch.reshape(nblk, 1, BLK_N)

    out = pl.pallas_call(
        _fused_kernel,
        grid=(nblk,),
        in_specs=[
            pl.BlockSpec((8, BLK_N), lambda g: (0, g)),
            pl.BlockSpec((1, 1, BLK_N), lambda g: (g, 0, 0)),
            pl.BlockSpec((8, D), lambda g: (0, 0)),
        ],
        out_specs=pl.BlockSpec((N_GRAPHS, SD), lambda g: (0, 0)),
        out_shape=jax.ShapeDtypeStruct((N_GRAPHS, SD), jnp.float32),
        compiler_params=pltpu.CompilerParams(
            vmem_limit_bytes=100 * 1024 * 1024),
    )(xt, batch3, v)

    return out.reshape(N_GRAPHS, BUMP_STEPS, NUM_THETAS, NUM_PHIS)


# speedup vs baseline: 10.6179x; 1.0182x over previous
import jax
import jax.numpy as jnp
import numpy as np
from jax.experimental import pallas as pl
from jax.experimental.pallas import tpu as pltpu

NUM_THETAS = 16
NUM_PHIS = 16
BUMP_STEPS = 16
RADIUS = 1.1
N_GRAPHS = 8
D = NUM_THETAS * NUM_PHIS
SD = BUMP_STEPS * D

BLK_N = 8192

_LIN = np.linspace(-RADIUS, RADIUS, BUMP_STEPS).astype(np.float32)


def _directions():
    theta = jnp.linspace(0.0, jnp.pi, NUM_THETAS)
    phi = jnp.linspace(0.0, 2.0 * jnp.pi, NUM_PHIS)
    mt, mp = jnp.meshgrid(theta, phi, indexing="ij")
    v = jnp.stack(
        [
            (jnp.sin(mt) * jnp.cos(mp)).reshape(-1),
            (jnp.sin(mt) * jnp.sin(mp)).reshape(-1),
            jnp.cos(mt).reshape(-1),
        ],
        axis=0,
    )
    return v.astype(jnp.float32)


def _fused_kernel(xt_ref, batch_ref, v_ref, out_ref):
    @pl.when(pl.program_id(0) == 0)
    def _init():
        out_ref[...] = jnp.zeros_like(out_ref)

    nh100 = jax.lax.dot_general(
        xt_ref[...], v_ref[...], (((0,), (0,)), ((), ())),
        preferred_element_type=jnp.float32,
    )

    parts = []
    for s in range(BUMP_STEPS):
        arg = float(100.0 * _LIN[s]) - nh100
        parts.append(jnp.tanh(arg))
    tanh_all = jnp.concatenate(parts, axis=1)

    b_ids = jax.lax.broadcasted_iota(jnp.int32, (N_GRAPHS, BLK_N), 0)
    onehot = (b_ids == batch_ref[0]).astype(jnp.bfloat16)
    seg = jax.lax.dot_general(
        onehot, tanh_all, (((1,), (0,)), ((), ())),
        preferred_element_type=jnp.float32,
    )
    count = jnp.sum(onehot.astype(jnp.float32), axis=1, keepdims=True)
    out_ref[...] += 0.5 * seg + 0.5 * count


def kernel(x, batch):
    n = x.shape[0]
    xt = jnp.zeros((8, n), dtype=jnp.float32).at[:3, :].set(x.T)
    v = jnp.zeros((8, D), dtype=jnp.float32).at[:3, :].set(100.0 * _directions())
    nblk = n // BLK_N
    batch3 = batch.reshape(nblk, 1, BLK_N)

    out = pl.pallas_call(
        _fused_kernel,
        grid=(nblk,),
        in_specs=[
            pl.BlockSpec((8, BLK_N), lambda g: (0, g)),
            pl.BlockSpec((1, 1, BLK_N), lambda g: (g, 0, 0)),
            pl.BlockSpec((8, D), lambda g: (0, 0)),
        ],
        out_specs=pl.BlockSpec((N_GRAPHS, SD), lambda g: (0, 0)),
        out_shape=jax.ShapeDtypeStruct((N_GRAPHS, SD), jnp.float32),
        compiler_params=pltpu.CompilerParams(
            vmem_limit_bytes=100 * 1024 * 1024),
    )(xt, batch3, v)

    return out.reshape(N_GRAPHS, BUMP_STEPS, NUM_THETAS, NUM_PHIS)
